# baseline (speedup 1.0000x reference)
import jax
import jax.numpy as jnp
from jax import lax
from jax.experimental import pallas as pl
from jax.experimental.pallas import tpu as pltpu
from jax.experimental.pallas import tpu_sc as plsc

NC = 2
NS = 16
L = 16
NW = NC * NS

C = 2048
K = 6
KP = 8
S = 7816


def _pad_body(conn_hbm, out_hbm, buf_v):
    n_cells = conn_hbm.shape[0]
    wid = lax.axis_index("s") * NC + lax.axis_index("c")
    n_blocks = (n_cells + C - 1) // C

    def do_block(t, _):
        b = wid + t * NW
        base = jnp.minimum(b * C, n_cells - C)
        pltpu.sync_copy(conn_hbm.at[pl.ds(base, C)], buf_v)
        pltpu.sync_copy(buf_v, out_hbm.at[pl.ds(base, C), pl.ds(0, K)])
        return ()

    my_blocks = (n_blocks - wid + NW - 1) // NW
    lax.fori_loop(0, my_blocks, do_block, ())


def _interp_body(cid_hbm, sf_hbm, conn_hbm, tab_hbm, out_hbm,
                 tab_sp, idx_v, conn_v, flat_v, val_v, sf_v, o_v,
                 sem_c, sem_v, sem_s):
    M = cid_hbm.shape[0]
    N = tab_hbm.shape[0] // NC

    core = lax.axis_index("c")
    sub = lax.axis_index("s")

    base_s = (sub * (N // NS)) // 8 * 8

    def stage(p, _):
        off = jnp.minimum(base_s + p * S, N - S)
        pltpu.sync_copy(tab_hbm.at[pl.ds(core * N + off, S)],
                        val_v.at[pl.ds(0, S)])
        pltpu.sync_copy(val_v.at[pl.ds(0, S)], tab_sp.at[pl.ds(off, S)])
        return ()

    lax.fori_loop(0, (N // NS + S - 1) // S + 1, stage, ())
    plsc.subcore_barrier()

    iota = lax.iota(jnp.int32, L)
    kconsts = [jnp.full((L,), k, jnp.int32) for k in range(K)]
    n_chunks = (M + C - 1) // C

    def do_chunk(t, _):
        i = sub + t * NS
        base = jnp.minimum(i * C, M - C)
        pltpu.sync_copy(cid_hbm.at[pl.ds(base, C)], idx_v)
        sfd = pltpu.async_copy(sf_hbm.at[pl.ds(base, C)], sf_v, sem_s)

        pltpu.async_copy(conn_hbm.at[idx_v], conn_v, sem_c).wait()

        for k in range(K):
            def flat_loop(j, _, k=k):
                q = j * L + iota
                flat_v[pl.ds(k * C + j * L, L)] = plsc.load_gather(
                    conn_v, [q, kconsts[k]]) - 1
                return ()
            lax.fori_loop(0, C // L, flat_loop, (), unroll=8)

        pltpu.async_copy(tab_sp.at[flat_v], val_v, sem_v).wait()
        sfd.wait()

        def group(g, _):
            q = g * L + iota
            acc = jnp.zeros((L,), jnp.float32)
            for k in range(K):
                w = plsc.load_gather(sf_v, [q, kconsts[k]])
                acc = acc + w * val_v[pl.ds(k * C + g * L, L)]
            o_v[pl.ds(g * L, L)] = acc
            return ()

        lax.fori_loop(0, C // L, group, (), unroll=4)
        pltpu.sync_copy(o_v, out_hbm.at[pl.ds(core * M + base, C)])
        return ()

    my_chunks = (n_chunks - sub + NS - 1) // NS
    lax.fori_loop(0, my_chunks, do_chunk, ())


def kernel(x, cell_id, nodal_values, shape_functions, connectivity):
    del x
    F, N, _ = nodal_values.shape
    M = cell_id.shape[0]
    n_cells = connectivity.shape[0]

    mesh = plsc.VectorSubcoreMesh(core_axis_name="c", subcore_axis_name="s")
    cparams = pltpu.CompilerParams(
        needs_layout_passes=False, use_tc_tiling_on_sc=False)

    pad = pl.kernel(
        _pad_body,
        out_type=jax.ShapeDtypeStruct((n_cells, KP), jnp.int32),
        mesh=mesh,
        compiler_params=cparams,
        scratch_types=[pltpu.VMEM((C, K), jnp.int32)],
    )
    conn8 = pad(connectivity)

    tables = nodal_values[:, :, 0].reshape(F * N)

    run = pl.kernel(
        _interp_body,
        out_type=jax.ShapeDtypeStruct((F * M,), jnp.float32),
        mesh=mesh,
        compiler_params=cparams,
        scratch_types=[
            pltpu.VMEM_SHARED((N,), jnp.float32),
            pltpu.VMEM((C,), jnp.int32),
            pltpu.VMEM((C, KP), jnp.int32),
            pltpu.VMEM((C * K,), jnp.int32),
            pltpu.VMEM((C * K,), jnp.float32),
            pltpu.VMEM((C, K), jnp.float32),
            pltpu.VMEM((C,), jnp.float32),
            pltpu.SemaphoreType.DMA,
            pltpu.SemaphoreType.DMA,
            pltpu.SemaphoreType.DMA,
        ],
    )
    return run(cell_id, shape_functions, conn8, tables).reshape(F, M)


# speedup vs baseline: 1.4250x; 1.4250x over previous
import jax
import jax.numpy as jnp
from jax import lax
from jax.experimental import pallas as pl
from jax.experimental.pallas import tpu as pltpu
from jax.experimental.pallas import tpu_sc as plsc

NC = 2
NS = 16
L = 16
NW = NC * NS

C = 2048
K = 6
KP = 8
S = 7816


def _interp_body(cid_hbm, sf_hbm, conn_hbm, tab_hbm, out_hbm,
                 tab_sp, idx_v, conn_v, flat_v, val_v, sf_v, o_v,
                 sem_c, sem_v, sem_s):
    M = cid_hbm.shape[0]
    N = tab_hbm.shape[0] // NC

    core = lax.axis_index("c")
    sub = lax.axis_index("s")

    base_s = (sub * (N // NS)) // 8 * 8

    def stage(p, _):
        off = jnp.minimum(base_s + p * S, N - S)
        pltpu.sync_copy(tab_hbm.at[pl.ds(core * N + off, S)],
                        val_v.at[pl.ds(0, S)])
        pltpu.sync_copy(val_v.at[pl.ds(0, S)], tab_sp.at[pl.ds(off, S)])
        return ()

    lax.fori_loop(0, (N // NS + S - 1) // S + 1, stage, ())
    plsc.subcore_barrier()

    iota = lax.iota(jnp.int32, L)
    kconsts = [jnp.full((L,), k, jnp.int32) for k in range(K)]
    n_chunks = (M + C - 1) // C

    def do_chunk(t, _):
        i = sub + t * NS
        base = jnp.minimum(i * C, M - C)
        pltpu.sync_copy(cid_hbm.at[pl.ds(base, C)], idx_v)
        sfd = pltpu.async_copy(sf_hbm.at[pl.ds(base, C)], sf_v, sem_s)

        pltpu.async_copy(conn_hbm.at[idx_v], conn_v, sem_c).wait()

        for k in range(K):
            def flat_loop(j, _, k=k):
                q = j * L + iota
                flat_v[pl.ds(k * C + j * L, L)] = plsc.load_gather(
                    conn_v, [q, kconsts[k]]) - 1
                return ()
            lax.fori_loop(0, C // L, flat_loop, (), unroll=8)

        pltpu.async_copy(tab_sp.at[flat_v], val_v, sem_v).wait()
        sfd.wait()

        def group(g, _):
            q = g * L + iota
            acc = jnp.zeros((L,), jnp.float32)
            for k in range(K):
                w = plsc.load_gather(sf_v, [q, kconsts[k]])
                acc = acc + w * val_v[pl.ds(k * C + g * L, L)]
            o_v[pl.ds(g * L, L)] = acc
            return ()

        lax.fori_loop(0, C // L, group, (), unroll=4)
        pltpu.sync_copy(o_v, out_hbm.at[pl.ds(core * M + base, C)])
        return ()

    my_chunks = (n_chunks - sub + NS - 1) // NS
    lax.fori_loop(0, my_chunks, do_chunk, ())


def kernel(x, cell_id, nodal_values, shape_functions, connectivity):
    del x
    F, N, _ = nodal_values.shape
    M = cell_id.shape[0]
    n_cells = connectivity.shape[0]

    mesh = plsc.VectorSubcoreMesh(core_axis_name="c", subcore_axis_name="s")
    cparams = pltpu.CompilerParams(
        needs_layout_passes=False, use_tc_tiling_on_sc=False)

    conn8 = jnp.concatenate(
        [connectivity, jnp.zeros((n_cells, KP - K), jnp.int32)], axis=1)

    tables = nodal_values[:, :, 0].reshape(F * N)

    run = pl.kernel(
        _interp_body,
        out_type=jax.ShapeDtypeStruct((F * M,), jnp.float32),
        mesh=mesh,
        compiler_params=cparams,
        scratch_types=[
            pltpu.VMEM_SHARED((N,), jnp.float32),
            pltpu.VMEM((C,), jnp.int32),
            pltpu.VMEM((C, KP), jnp.int32),
            pltpu.VMEM((C * K,), jnp.int32),
            pltpu.VMEM((C * K,), jnp.float32),
            pltpu.VMEM((C, K), jnp.float32),
            pltpu.VMEM((C,), jnp.float32),
            pltpu.SemaphoreType.DMA,
            pltpu.SemaphoreType.DMA,
            pltpu.SemaphoreType.DMA,
        ],
    )
    return run(cell_id, shape_functions, conn8, tables).reshape(F, M)
